# baseline (speedup 1.0000x reference)
import functools

import jax
import jax.numpy as jnp
from jax import lax
from jax.experimental import pallas as pl
from jax.experimental.pallas import tpu as pltpu
from jax.experimental.pallas import tpu_sc as plsc

N = 10000
E = 320000
D = 128
EPS = 1e-08

NC = 2
NS = 16
NW = NC * NS

EPT = E // NW
CH = 128
NCHUNK = 80
EPTP = NCHUNK * CH
PAD = EPTP - EPT
NBLK = NCHUNK // 8
TRASH = N
ACC_ROWS = N + 16 * NS
ZR = 16
NZB = N // ZR
ZPT = (NZB + NS - 1) // NS


def _mesh():
    return plsc.VectorSubcoreMesh(
        core_axis_name="c", subcore_axis_name="s",
        num_cores=NC, num_subcores=NS)


def _deg_body(dst_hbm, degp_hbm, idx_v, deg_v):
    c = lax.axis_index("c")
    s = lax.axis_index("s")
    wid = c * NS + s
    pltpu.sync_copy(dst_hbm.at[pl.ds(wid * EPT, EPT)], idx_v)

    zeros16 = jnp.zeros((16,), jnp.int32)

    def zbody(i, _):
        deg_v[pl.ds(i * 16, 16)] = zeros16
        return 0

    lax.fori_loop(0, N // 16, zbody, 0)

    ones16 = jnp.ones((16,), jnp.int32)

    def body(i, _):
        idx = idx_v[pl.ds(i * 16, 16)]
        plsc.addupdate_scatter(deg_v, [idx], ones16)
        return 0

    lax.fori_loop(0, EPT // 16, body, 0)
    pltpu.sync_copy(deg_v, degp_hbm.at[wid])


def _deg_call(dst):
    return pl.kernel(
        _deg_body,
        out_type=jax.ShapeDtypeStruct((NW, N), jnp.int32),
        mesh=_mesh(),
        scratch_types=[
            pltpu.VMEM((EPT,), jnp.int32),
            pltpu.VMEM((N,), jnp.int32),
        ],
        compiler_params=pltpu.CompilerParams(needs_layout_passes=False),
    )(dst)


def _scat_body(hs_hbm, src_hbm, dst_hbm, out_hbm,
               acc_sh, srcfull, dstv0, dstv1, rows0, rows1,
               g0, g1, d0, d1):
    c = lax.axis_index("c")
    s = lax.axis_index("s")
    wid = c * NS + s
    rows = [rows0, rows1]
    gsems = [g0, g1]
    dstv = [dstv0, dstv1]

    zeros16f = jnp.zeros((16,), jnp.float32)

    def zinit(i, _):
        rows0[i // (D // 16), pl.ds((i % (D // 16)) * 16, 16)] = zeros16f
        return 0

    lax.fori_loop(0, CH * (D // 16), zinit, 0)

    def zbody(t, _):
        i = t * NS + s

        @pl.when(i < NZB)
        def _():
            pltpu.sync_copy(rows0.at[pl.ds(0, ZR)], acc_sh.at[pl.ds(i * ZR, ZR)])
        return 0

    lax.fori_loop(0, ZPT, zbody, 0)
    plsc.subcore_barrier()

    pltpu.sync_copy(src_hbm.at[wid], srcfull)

    def dstload(blk, slot, sem):
        return pltpu.make_async_copy(dst_hbm.at[wid].at[blk], dstv[slot], sem)

    HC = CH // 2

    class _Pair:
        def __init__(self, a, b):
            self.a, self.b = a, b

        def start(self):
            self.a.start()
            self.b.start()

        def wait(self):
            self.a.wait()
            self.b.wait()

    def gather(j, par):
        return _Pair(
            pltpu.make_async_copy(
                hs_hbm.at[srcfull.at[pl.ds(j * CH, HC)]],
                rows[par].at[pl.ds(0, HC)], gsems[par]),
            pltpu.make_async_copy(
                hs_hbm.at[srcfull.at[pl.ds(j * CH + HC, HC)]],
                rows[par].at[pl.ds(HC, HC)], gsems[par]))

    dstload(0, 0, d0).start()
    dstload(1, 1, d1).start()
    gather(0, 0).start()

    def body(sb, _):
        base = sb * 16
        for jj in range(16):
            j = base + jj
            par = jj % 2
            if jj == 0:
                dstload(0, 0, d0).wait()
            if jj == 8:
                dstload(0, 1, d1).wait()
            if jj < 15:
                gather(j + 1, 1 - par).start()
            else:
                @pl.when(sb < NBLK // 2 - 1)
                def _():
                    gather(j + 1, 1 - par).start()
            gather(j, par).wait()
            pltpu.sync_copy(rows[par], acc_sh.at[dstv[jj // 8].at[jj % 8]],
                            add=True)
            if jj == 7:
                @pl.when(sb < NBLK // 2 - 1)
                def _():
                    dstload(2 * sb + 2, 0, d0).start()
            if jj == 15:
                @pl.when(sb < NBLK // 2 - 1)
                def _():
                    dstload(2 * sb + 3, 1, d1).start()
        return 0

    lax.fori_loop(0, NBLK // 2, body, 0)
    plsc.subcore_barrier()

    def wbody(t, _):
        i = t * NS + s

        @pl.when(i < NZB)
        def _():
            pltpu.sync_copy(acc_sh.at[pl.ds(i * ZR, ZR)],
                            out_hbm.at[c].at[pl.ds(i * ZR, ZR)])
        return 0

    lax.fori_loop(0, ZPT, wbody, 0)


def _scat_call(hs, srcp, dstp):
    return pl.kernel(
        _scat_body,
        out_type=jax.ShapeDtypeStruct((NC, N, D), jnp.float32),
        mesh=_mesh(),
        scratch_types=[
            pltpu.VMEM_SHARED((ACC_ROWS, D), jnp.float32),
            pltpu.VMEM((EPTP,), jnp.int32),
            pltpu.VMEM((8, CH), jnp.int32),
            pltpu.VMEM((8, CH), jnp.int32),
            pltpu.VMEM((CH, D), jnp.float32),
            pltpu.VMEM((CH, D), jnp.float32),
            pltpu.SemaphoreType.DMA,
            pltpu.SemaphoreType.DMA,
            pltpu.SemaphoreType.DMA,
            pltpu.SemaphoreType.DMA,
        ],
        compiler_params=pltpu.CompilerParams(needs_layout_passes=False),
    )(hs, srcp, dstp)


BR = 2000


def _pre_body(x_ref, w_ref, degp_ref, hs_ref):
    deg = jnp.sum(degp_ref[...], axis=1).astype(jnp.float32) + 1.0
    dinv = lax.rsqrt(deg)
    h = jnp.dot(x_ref[...], w_ref[...], preferred_element_type=jnp.float32)
    hs_ref[...] = h * dinv[:, None]


def _pre_call(x, W, degp_t):
    grid = (N // BR,)
    return pl.pallas_call(
        _pre_body,
        grid=grid,
        in_specs=[
            pl.BlockSpec((BR, D), lambda i: (i, 0)),
            pl.BlockSpec((D, D), lambda i: (0, 0)),
            pl.BlockSpec((BR, NW), lambda i: (i, 0)),
        ],
        out_specs=pl.BlockSpec((BR, D), lambda i: (i, 0)),
        out_shape=jax.ShapeDtypeStruct((N, D), jnp.float32),
    )(x, W, degp_t)


def _post_body(s01_ref, hs_ref, x_ref, degp_ref, b_ref, g_ref, bt_ref,
               out_ref):
    deg = jnp.sum(degp_ref[...], axis=1).astype(jnp.float32) + 1.0
    dinv = lax.rsqrt(deg)
    ssum = s01_ref[0] + s01_ref[1]
    pre = dinv[:, None] * (ssum + hs_ref[...]) + b_ref[...]
    h = jnp.maximum(pre, 0.0) + x_ref[...]
    mean = jnp.mean(h, axis=-1, keepdims=True)
    hc = h - mean
    var = jnp.mean(hc * hc, axis=-1, keepdims=True)
    out_ref[...] = hc * lax.rsqrt(var + EPS) * g_ref[...] + bt_ref[...]


def _post_call(s01, hs, x, degp_t, b, gamma, beta):
    grid = (N // BR,)
    return pl.pallas_call(
        _post_body,
        grid=grid,
        in_specs=[
            pl.BlockSpec((2, BR, D), lambda i: (0, i, 0)),
            pl.BlockSpec((BR, D), lambda i: (i, 0)),
            pl.BlockSpec((BR, D), lambda i: (i, 0)),
            pl.BlockSpec((BR, NW), lambda i: (i, 0)),
            pl.BlockSpec((1, D), lambda i: (0, 0)),
            pl.BlockSpec((1, D), lambda i: (0, 0)),
            pl.BlockSpec((1, D), lambda i: (0, 0)),
        ],
        out_specs=pl.BlockSpec((BR, D), lambda i: (i, 0)),
        out_shape=jax.ShapeDtypeStruct((N, D), jnp.float32),
    )(s01, hs, x, degp_t, b[None, :], gamma[None, :], beta[None, :])


def kernel(x, edge_index, W, b, gamma, beta):
    src = edge_index[0].astype(jnp.int32)
    dst = edge_index[1].astype(jnp.int32)
    pad_src = (jnp.arange(NW * PAD, dtype=jnp.int32) * 37 % N).reshape(NW, PAD)
    srcp = jnp.concatenate([src.reshape(NW, EPT), pad_src], axis=1)
    trash = (TRASH + (jnp.arange(NW, dtype=jnp.int32) % NS)[:, None] * 16
             + jnp.arange(PAD, dtype=jnp.int32)[None, :] % 16)
    dstp = jnp.concatenate(
        [dst.reshape(NW, EPT), trash], axis=1).reshape(NW, NBLK, 8, CH)

    degp_t = jnp.transpose(_deg_call(dst))
    hs = _pre_call(x, W, degp_t)
    s01 = _scat_call(hs, srcp, dstp)
    return _post_call(s01, hs, x, degp_t, b, gamma, beta)


# speedup vs baseline: 1.0200x; 1.0200x over previous
import functools

import jax
import jax.numpy as jnp
from jax import lax
from jax.experimental import pallas as pl
from jax.experimental.pallas import tpu as pltpu
from jax.experimental.pallas import tpu_sc as plsc

N = 10000
E = 320000
D = 128
EPS = 1e-08

NC = 2
NS = 16
NW = NC * NS

EPT = E // NW
CH = 128
NCHUNK = 80
EPTP = NCHUNK * CH
PAD = EPTP - EPT
NBLK = NCHUNK // 8
TRASH = N
ACC_ROWS = N + 16 * NS
ZR = 16
NZB = N // ZR
ZPT = (NZB + NS - 1) // NS


def _mesh():
    return plsc.VectorSubcoreMesh(
        core_axis_name="c", subcore_axis_name="s",
        num_cores=NC, num_subcores=NS)


def _deg_body(dst_hbm, degp_hbm, idx_v, deg_v):
    c = lax.axis_index("c")
    s = lax.axis_index("s")
    wid = c * NS + s
    pltpu.sync_copy(dst_hbm.at[pl.ds(wid * EPT, EPT)], idx_v)

    zeros16 = jnp.zeros((16,), jnp.int32)

    def zbody(i, _):
        deg_v[pl.ds(i * 16, 16)] = zeros16
        return 0

    lax.fori_loop(0, N // 16, zbody, 0)

    ones16 = jnp.ones((16,), jnp.int32)

    def body(i, _):
        idx = idx_v[pl.ds(i * 16, 16)]
        plsc.addupdate_scatter(deg_v, [idx], ones16)
        return 0

    lax.fori_loop(0, EPT // 16, body, 0)
    pltpu.sync_copy(deg_v, degp_hbm.at[wid])


def _deg_call(dst):
    return pl.kernel(
        _deg_body,
        out_type=jax.ShapeDtypeStruct((NW, N), jnp.int32),
        mesh=_mesh(),
        scratch_types=[
            pltpu.VMEM((EPT,), jnp.int32),
            pltpu.VMEM((N,), jnp.int32),
        ],
        compiler_params=pltpu.CompilerParams(needs_layout_passes=False),
    )(dst)


def _scat_body(hs_hbm, src_hbm, dst_hbm, out_hbm,
               acc_sh, srcfull, dstv0, dstv1, rows0, rows1,
               g0, g1, d0, d1, s0, s1):
    c = lax.axis_index("c")
    s = lax.axis_index("s")
    wid = c * NS + s
    rows = [rows0, rows1]
    gsems = [g0, g1]
    ssems = [s0, s1]
    dstv = [dstv0, dstv1]

    zeros16f = jnp.zeros((16,), jnp.float32)

    def zinit(i, _):
        rows0[i // (D // 16), pl.ds((i % (D // 16)) * 16, 16)] = zeros16f
        return 0

    lax.fori_loop(0, CH * (D // 16), zinit, 0)

    def zbody(t, _):
        i = t * NS + s

        @pl.when(i < NZB)
        def _():
            pltpu.sync_copy(rows0.at[pl.ds(0, ZR)], acc_sh.at[pl.ds(i * ZR, ZR)])
        return 0

    lax.fori_loop(0, ZPT, zbody, 0)
    plsc.subcore_barrier()

    pltpu.sync_copy(src_hbm.at[wid], srcfull)

    def dstload(blk, slot, sem):
        return pltpu.make_async_copy(dst_hbm.at[wid].at[blk], dstv[slot], sem)

    def gather(j, par):
        return pltpu.make_async_copy(
            hs_hbm.at[srcfull.at[pl.ds(j * CH, CH)]], rows[par], gsems[par])

    def scat_start(par, blkslot, row):
        pltpu.async_copy(rows[par], acc_sh.at[dstv[blkslot].at[row]],
                         ssems[par], add=True)

    def scat_wait(par):
        pltpu.make_async_copy(rows[par], acc_sh.at[dstv[0].at[0]],
                              ssems[par]).wait()

    dstload(0, 0, d0).start()
    dstload(1, 1, d1).start()
    gather(0, 0).start()

    def body(sb, _):
        base = sb * 16
        for jj in range(16):
            j = base + jj
            par = jj % 2
            q = 1 - par
            if jj == 0:
                dstload(0, 0, d0).wait()

                @pl.when(sb > 0)
                def _():
                    scat_wait(q)
                gather(j + 1, q).start()

                @pl.when(sb > 0)
                def _():
                    dstload(2 * sb + 1, 1, d1).start()
            elif jj < 15:
                if jj == 8:
                    dstload(0, 1, d1).wait()
                scat_wait(q)
                gather(j + 1, q).start()
            else:
                @pl.when(sb < NBLK // 2 - 1)
                def _():
                    scat_wait(q)
                    gather(j + 1, q).start()
            gather(j, par).wait()
            scat_start(par, jj // 8, jj % 8)
            if jj == 8:
                @pl.when(sb < NBLK // 2 - 1)
                def _():
                    dstload(2 * sb + 2, 0, d0).start()
        return 0

    lax.fori_loop(0, NBLK // 2, body, 0)
    scat_wait(0)
    scat_wait(1)
    plsc.subcore_barrier()

    def wbody(t, _):
        i = t * NS + s

        @pl.when(i < NZB)
        def _():
            pltpu.sync_copy(acc_sh.at[pl.ds(i * ZR, ZR)],
                            out_hbm.at[c].at[pl.ds(i * ZR, ZR)])
        return 0

    lax.fori_loop(0, ZPT, wbody, 0)


def _scat_call(hs, srcp, dstp):
    return pl.kernel(
        _scat_body,
        out_type=jax.ShapeDtypeStruct((NC, N, D), jnp.float32),
        mesh=_mesh(),
        scratch_types=[
            pltpu.VMEM_SHARED((ACC_ROWS, D), jnp.float32),
            pltpu.VMEM((EPTP,), jnp.int32),
            pltpu.VMEM((8, CH), jnp.int32),
            pltpu.VMEM((8, CH), jnp.int32),
            pltpu.VMEM((CH, D), jnp.float32),
            pltpu.VMEM((CH, D), jnp.float32),
            pltpu.SemaphoreType.DMA,
            pltpu.SemaphoreType.DMA,
            pltpu.SemaphoreType.DMA,
            pltpu.SemaphoreType.DMA,
            pltpu.SemaphoreType.DMA,
            pltpu.SemaphoreType.DMA,
        ],
        compiler_params=pltpu.CompilerParams(needs_layout_passes=False),
    )(hs, srcp, dstp)


BR = 2000


def _pre_body(x_ref, w_ref, degp_ref, hs_ref):
    deg = jnp.sum(degp_ref[...], axis=1).astype(jnp.float32) + 1.0
    dinv = lax.rsqrt(deg)
    h = jnp.dot(x_ref[...], w_ref[...], preferred_element_type=jnp.float32)
    hs_ref[...] = h * dinv[:, None]


def _pre_call(x, W, degp_t):
    grid = (N // BR,)
    return pl.pallas_call(
        _pre_body,
        grid=grid,
        in_specs=[
            pl.BlockSpec((BR, D), lambda i: (i, 0)),
            pl.BlockSpec((D, D), lambda i: (0, 0)),
            pl.BlockSpec((BR, NW), lambda i: (i, 0)),
        ],
        out_specs=pl.BlockSpec((BR, D), lambda i: (i, 0)),
        out_shape=jax.ShapeDtypeStruct((N, D), jnp.float32),
    )(x, W, degp_t)


def _post_body(s01_ref, hs_ref, x_ref, degp_ref, b_ref, g_ref, bt_ref,
               out_ref):
    deg = jnp.sum(degp_ref[...], axis=1).astype(jnp.float32) + 1.0
    dinv = lax.rsqrt(deg)
    ssum = s01_ref[0] + s01_ref[1]
    pre = dinv[:, None] * (ssum + hs_ref[...]) + b_ref[...]
    h = jnp.maximum(pre, 0.0) + x_ref[...]
    mean = jnp.mean(h, axis=-1, keepdims=True)
    hc = h - mean
    var = jnp.mean(hc * hc, axis=-1, keepdims=True)
    out_ref[...] = hc * lax.rsqrt(var + EPS) * g_ref[...] + bt_ref[...]


def _post_call(s01, hs, x, degp_t, b, gamma, beta):
    grid = (N // BR,)
    return pl.pallas_call(
        _post_body,
        grid=grid,
        in_specs=[
            pl.BlockSpec((2, BR, D), lambda i: (0, i, 0)),
            pl.BlockSpec((BR, D), lambda i: (i, 0)),
            pl.BlockSpec((BR, D), lambda i: (i, 0)),
            pl.BlockSpec((BR, NW), lambda i: (i, 0)),
            pl.BlockSpec((1, D), lambda i: (0, 0)),
            pl.BlockSpec((1, D), lambda i: (0, 0)),
            pl.BlockSpec((1, D), lambda i: (0, 0)),
        ],
        out_specs=pl.BlockSpec((BR, D), lambda i: (i, 0)),
        out_shape=jax.ShapeDtypeStruct((N, D), jnp.float32),
    )(s01, hs, x, degp_t, b[None, :], gamma[None, :], beta[None, :])


def kernel(x, edge_index, W, b, gamma, beta):
    src = edge_index[0].astype(jnp.int32)
    dst = edge_index[1].astype(jnp.int32)
    pad_src = (jnp.arange(NW * PAD, dtype=jnp.int32) * 37 % N).reshape(NW, PAD)
    srcp = jnp.concatenate([src.reshape(NW, EPT), pad_src], axis=1)
    trash = (TRASH + (jnp.arange(NW, dtype=jnp.int32) % NS)[:, None] * 16
             + jnp.arange(PAD, dtype=jnp.int32)[None, :] % 16)
    dstp = jnp.concatenate(
        [dst.reshape(NW, EPT), trash], axis=1).reshape(NW, NBLK, 8, CH)

    degp_t = jnp.transpose(_deg_call(dst))
    hs = _pre_call(x, W, degp_t)
    s01 = _scat_call(hs, srcp, dstp)
    return _post_call(s01, hs, x, degp_t, b, gamma, beta)


# speedup vs baseline: 1.0214x; 1.0014x over previous
import functools

import jax
import jax.numpy as jnp
from jax import lax
from jax.experimental import pallas as pl
from jax.experimental.pallas import tpu as pltpu
from jax.experimental.pallas import tpu_sc as plsc

N = 10000
E = 320000
D = 128
EPS = 1e-08

NC = 2
NS = 16
NW = NC * NS

EPT = E // NW
CH = 128
NCHUNK = 80
EPTP = NCHUNK * CH
PAD = EPTP - EPT
NBLK = NCHUNK // 8
TRASH = N
ACC_ROWS = N + 16 * NS
ZR = 16
NZB = N // ZR
ZPT = (NZB + NS - 1) // NS


def _mesh():
    return plsc.VectorSubcoreMesh(
        core_axis_name="c", subcore_axis_name="s",
        num_cores=NC, num_subcores=NS)


def _deg_body(dst_hbm, degp_hbm, idx_v, deg_v):
    c = lax.axis_index("c")
    s = lax.axis_index("s")
    wid = c * NS + s
    pltpu.sync_copy(dst_hbm.at[pl.ds(wid * EPT, EPT)], idx_v)

    zeros16 = jnp.zeros((16,), jnp.int32)

    def zbody(i, _):
        deg_v[pl.ds(i * 16, 16)] = zeros16
        return 0

    lax.fori_loop(0, N // 16, zbody, 0)

    ones16 = jnp.ones((16,), jnp.int32)

    def body(i, _):
        idx = idx_v[pl.ds(i * 16, 16)]
        plsc.addupdate_scatter(deg_v, [idx], ones16)
        return 0

    lax.fori_loop(0, EPT // 16, body, 0)
    pltpu.sync_copy(deg_v, degp_hbm.at[wid])


def _deg_call(dst):
    return pl.kernel(
        _deg_body,
        out_type=jax.ShapeDtypeStruct((NW, N), jnp.int32),
        mesh=_mesh(),
        scratch_types=[
            pltpu.VMEM((EPT,), jnp.int32),
            pltpu.VMEM((N,), jnp.int32),
        ],
        compiler_params=pltpu.CompilerParams(needs_layout_passes=False),
    )(dst)


def _scat_body(hs_hbm, src_hbm, dst_hbm, out_hbm,
               acc_sh, srcfull, dstv0, dstv1, rows0, rows1,
               g0, g1, d0, d1):
    c = lax.axis_index("c")
    s = lax.axis_index("s")
    wid = c * NS + s
    rows = [rows0, rows1]
    gsems = [g0, g1]
    dstv = [dstv0, dstv1]

    zeros16f = jnp.zeros((16,), jnp.float32)

    def zinit(i, _):
        rows0[i // (D // 16), pl.ds((i % (D // 16)) * 16, 16)] = zeros16f
        return 0

    lax.fori_loop(0, CH * (D // 16), zinit, 0)

    def zbody(t, _):
        i = t * NS + s

        @pl.when(i < NZB)
        def _():
            pltpu.sync_copy(rows0.at[pl.ds(0, ZR)], acc_sh.at[pl.ds(i * ZR, ZR)])
        return 0

    lax.fori_loop(0, ZPT, zbody, 0)
    plsc.subcore_barrier()

    pltpu.sync_copy(src_hbm.at[wid], srcfull)

    def dstload(blk, slot, sem):
        return pltpu.make_async_copy(dst_hbm.at[wid].at[blk], dstv[slot], sem)

    def gather(j, par):
        return pltpu.make_async_copy(
            hs_hbm.at[srcfull.at[pl.ds(j * CH, CH)]], rows[par], gsems[par])

    dstload(0, 0, d0).start()
    dstload(1, 1, d1).start()
    gather(0, 0).start()

    def body(sb, _):
        base = sb * 16
        for jj in range(16):
            j = base + jj
            par = jj % 2
            if jj == 0:
                dstload(0, 0, d0).wait()
            if jj == 8:
                dstload(0, 1, d1).wait()
            if jj < 15:
                gather(j + 1, 1 - par).start()
            else:
                @pl.when(sb < NBLK // 2 - 1)
                def _():
                    gather(j + 1, 1 - par).start()
            gather(j, par).wait()
            pltpu.sync_copy(rows[par], acc_sh.at[dstv[jj // 8].at[jj % 8]],
                            add=True)
            if jj == 7:
                @pl.when(sb < NBLK // 2 - 1)
                def _():
                    dstload(2 * sb + 2, 0, d0).start()
            if jj == 15:
                @pl.when(sb < NBLK // 2 - 1)
                def _():
                    dstload(2 * sb + 3, 1, d1).start()
        return 0

    lax.fori_loop(0, NBLK // 2, body, 0)
    plsc.subcore_barrier()

    def wbody(t, _):
        i = t * NS + s

        @pl.when(i < NZB)
        def _():
            pltpu.sync_copy(acc_sh.at[pl.ds(i * ZR, ZR)],
                            out_hbm.at[c].at[pl.ds(i * ZR, ZR)])
        return 0

    lax.fori_loop(0, ZPT, wbody, 0)


def _scat_call(hs, srcp, dstp):
    return pl.kernel(
        _scat_body,
        out_type=jax.ShapeDtypeStruct((NC, N, D), jnp.float32),
        mesh=_mesh(),
        scratch_types=[
            pltpu.VMEM_SHARED((ACC_ROWS, D), jnp.float32),
            pltpu.VMEM((EPTP,), jnp.int32),
            pltpu.VMEM((8, CH), jnp.int32),
            pltpu.VMEM((8, CH), jnp.int32),
            pltpu.VMEM((CH, D), jnp.float32),
            pltpu.VMEM((CH, D), jnp.float32),
            pltpu.SemaphoreType.DMA,
            pltpu.SemaphoreType.DMA,
            pltpu.SemaphoreType.DMA,
            pltpu.SemaphoreType.DMA,
        ],
        compiler_params=pltpu.CompilerParams(needs_layout_passes=False),
    )(hs, srcp, dstp)


BR = 2000


def _pre_body(x_ref, w_ref, degp_ref, hs_ref):
    deg = jnp.sum(degp_ref[...], axis=1).astype(jnp.float32) + 1.0
    dinv = lax.rsqrt(deg)
    h = jnp.dot(x_ref[...], w_ref[...], preferred_element_type=jnp.float32)
    hs_ref[...] = h * dinv[:, None]


def _pre_call(x, W, degp_t):
    grid = (N // BR,)
    return pl.pallas_call(
        _pre_body,
        grid=grid,
        in_specs=[
            pl.BlockSpec((BR, D), lambda i: (i, 0)),
            pl.BlockSpec((D, D), lambda i: (0, 0)),
            pl.BlockSpec((BR, NW), lambda i: (i, 0)),
        ],
        out_specs=pl.BlockSpec((BR, D), lambda i: (i, 0)),
        out_shape=jax.ShapeDtypeStruct((N, D), jnp.float32),
    )(x, W, degp_t)


def _post_body(s01_ref, hs_ref, x_ref, degp_ref, b_ref, g_ref, bt_ref,
               out_ref):
    deg = jnp.sum(degp_ref[...], axis=1).astype(jnp.float32) + 1.0
    dinv = lax.rsqrt(deg)
    ssum = s01_ref[0] + s01_ref[1]
    pre = dinv[:, None] * (ssum + hs_ref[...]) + b_ref[...]
    h = jnp.maximum(pre, 0.0) + x_ref[...]
    mean = jnp.mean(h, axis=-1, keepdims=True)
    hc = h - mean
    var = jnp.mean(hc * hc, axis=-1, keepdims=True)
    out_ref[...] = hc * lax.rsqrt(var + EPS) * g_ref[...] + bt_ref[...]


def _post_call(s01, hs, x, degp_t, b, gamma, beta):
    grid = (N // BR,)
    return pl.pallas_call(
        _post_body,
        grid=grid,
        in_specs=[
            pl.BlockSpec((2, BR, D), lambda i: (0, i, 0)),
            pl.BlockSpec((BR, D), lambda i: (i, 0)),
            pl.BlockSpec((BR, D), lambda i: (i, 0)),
            pl.BlockSpec((BR, NW), lambda i: (i, 0)),
            pl.BlockSpec((1, D), lambda i: (0, 0)),
            pl.BlockSpec((1, D), lambda i: (0, 0)),
            pl.BlockSpec((1, D), lambda i: (0, 0)),
        ],
        out_specs=pl.BlockSpec((BR, D), lambda i: (i, 0)),
        out_shape=jax.ShapeDtypeStruct((N, D), jnp.float32),
    )(s01, hs, x, degp_t, b[None, :], gamma[None, :], beta[None, :])


def kernel(x, edge_index, W, b, gamma, beta):
    src = edge_index[0].astype(jnp.int32)
    dst = edge_index[1].astype(jnp.int32)
    pad_src = (jnp.arange(NW * PAD, dtype=jnp.int32) * 37 % N).reshape(NW, PAD)
    srcp = jnp.concatenate([src.reshape(NW, EPT), pad_src], axis=1)
    trash = (TRASH + (jnp.arange(NW, dtype=jnp.int32) % NS)[:, None] * 16
             + jnp.arange(PAD, dtype=jnp.int32)[None, :] % 16)
    dstp = jnp.concatenate(
        [dst.reshape(NW, EPT), trash], axis=1).reshape(NW, NBLK, 8, CH)

    degp_t = jnp.transpose(_deg_call(dst))
    hs = _pre_call(x, W, degp_t)
    s01 = _scat_call(hs, srcp, dstp)
    return _post_call(s01, hs, x, degp_t, b, gamma, beta)
